# baseline (speedup 1.0000x reference)
import functools

import jax
import jax.numpy as jnp
from jax import lax
from jax.experimental import pallas as pl
from jax.experimental.pallas import tpu as pltpu
from jax.experimental.pallas import tpu_sc as plsc

B = 16384
NUM_SUBCORES = 16
LANES = 16
B_PER_W = B // NUM_SUBCORES
NCHUNK = 1
CHUNK = B_PER_W // NCHUNK

_mesh = plsc.VectorSubcoreMesh(
    core_axis_name="c", subcore_axis_name="s", num_cores=1
)


@functools.partial(
    pl.kernel,
    out_type=jax.ShapeDtypeStruct((B,), jnp.float32),
    mesh=_mesh,
    compiler_params=pltpu.CompilerParams(needs_layout_passes=False),
    scratch_types=[
        pltpu.VMEM((B_PER_W,), jnp.int32),
        pltpu.VMEM((2,), jnp.float32),
        pltpu.VMEM((B_PER_W,), jnp.float32),
        pltpu.SemaphoreType.DMA((NCHUNK,)),
        pltpu.SemaphoreType.DMA((NCHUNK,)),
        pltpu.SemaphoreType.DMA,
    ],
)
def _lookup(idx_hbm, tab_hbm, out_hbm, idx_v, tab_v, out_v, isem, osem, tsem):
    base = lax.axis_index("s") * B_PER_W
    cp_t = pltpu.async_copy(tab_hbm, tab_v, tsem)
    in_cps = [
        pltpu.async_copy(
            idx_hbm.at[pl.ds(base + c * CHUNK, CHUNK)],
            idx_v.at[pl.ds(c * CHUNK, CHUNK)],
            isem.at[c],
        )
        for c in range(NCHUNK)
    ]
    cp_t.wait()
    out_cps = []
    for c in range(NCHUNK):
        in_cps[c].wait()

        @pl.loop(0, CHUNK, step=LANES)
        def _(i, c=c):
            off = c * CHUNK + i
            out_v[pl.ds(off, LANES)] = plsc.load_gather(
                tab_v, [idx_v[pl.ds(off, LANES)]]
            )

        out_cps.append(
            pltpu.async_copy(
                out_v.at[pl.ds(c * CHUNK, CHUNK)],
                out_hbm.at[pl.ds(base + c * CHUNK, CHUNK)],
                osem.at[c],
            )
        )
    for cp in out_cps:
        cp.wait()


def kernel(indices, table):
    out = _lookup(indices.astype(jnp.int32), table.reshape(2))
    return out.reshape(B, 1)


# speedup vs baseline: 1.0178x; 1.0178x over previous
import functools

import jax
import jax.numpy as jnp
from jax import lax
from jax.experimental import pallas as pl
from jax.experimental.pallas import tpu as pltpu
from jax.experimental.pallas import tpu_sc as plsc

B = 16384
NUM_SUBCORES = 16
LANES = 16
B_PER_W = B // NUM_SUBCORES
NCHUNK = 1
CHUNK = B_PER_W // NCHUNK

_mesh = plsc.VectorSubcoreMesh(
    core_axis_name="c", subcore_axis_name="s", num_cores=1
)


@functools.partial(
    pl.kernel,
    out_type=jax.ShapeDtypeStruct((B,), jnp.float32),
    mesh=_mesh,
    compiler_params=pltpu.CompilerParams(needs_layout_passes=False),
    scratch_types=[
        pltpu.VMEM((B_PER_W,), jnp.int32),
        pltpu.VMEM((2,), jnp.float32),
        pltpu.VMEM((B_PER_W,), jnp.float32),
        pltpu.SemaphoreType.DMA((NCHUNK,)),
        pltpu.SemaphoreType.DMA((NCHUNK,)),
        pltpu.SemaphoreType.DMA,
    ],
)
def _lookup(idx_hbm, tab_hbm, out_hbm, idx_v, tab_v, out_v, isem, osem, tsem):
    base = lax.axis_index("s") * B_PER_W
    cp_t = pltpu.async_copy(tab_hbm, tab_v, tsem)
    in_cps = [
        pltpu.async_copy(
            idx_hbm.at[pl.ds(base + c * CHUNK, CHUNK)],
            idx_v.at[pl.ds(c * CHUNK, CHUNK)],
            isem.at[c],
        )
        for c in range(NCHUNK)
    ]
    cp_t.wait()
    out_cps = []
    for c in range(NCHUNK):
        in_cps[c].wait()

        @plsc.parallel_loop(c * CHUNK, (c + 1) * CHUNK, step=LANES, unroll=4)
        def _(off):
            out_v[pl.ds(off, LANES)] = plsc.load_gather(
                tab_v, [idx_v[pl.ds(off, LANES)]]
            )

        out_cps.append(
            pltpu.async_copy(
                out_v.at[pl.ds(c * CHUNK, CHUNK)],
                out_hbm.at[pl.ds(base + c * CHUNK, CHUNK)],
                osem.at[c],
            )
        )
    for cp in out_cps:
        cp.wait()


def kernel(indices, table):
    out = _lookup(indices.astype(jnp.int32), table.reshape(2))
    return out.reshape(B, 1)


# speedup vs baseline: 1.0215x; 1.0036x over previous
import functools

import jax
import jax.numpy as jnp
from jax import lax
from jax.experimental import pallas as pl
from jax.experimental.pallas import tpu as pltpu
from jax.experimental.pallas import tpu_sc as plsc

B = 16384
NUM_SUBCORES = 16
LANES = 16
B_PER_W = B // NUM_SUBCORES
NCHUNK = 1
CHUNK = B_PER_W // NCHUNK

_mesh = plsc.VectorSubcoreMesh(
    core_axis_name="c", subcore_axis_name="s", num_cores=1
)


@functools.partial(
    pl.kernel,
    out_type=jax.ShapeDtypeStruct((B,), jnp.float32),
    mesh=_mesh,
    compiler_params=pltpu.CompilerParams(needs_layout_passes=False),
    scratch_types=[
        pltpu.VMEM((B_PER_W,), jnp.int32),
        pltpu.VMEM((2,), jnp.float32),
        pltpu.VMEM((B_PER_W,), jnp.float32),
        pltpu.SemaphoreType.DMA((NCHUNK,)),
        pltpu.SemaphoreType.DMA((NCHUNK,)),
        pltpu.SemaphoreType.DMA,
    ],
)
def _lookup(idx_hbm, tab_hbm, out_hbm, idx_v, tab_v, out_v, isem, osem, tsem):
    base = lax.axis_index("s") * B_PER_W
    cp_t = pltpu.async_copy(tab_hbm, tab_v, tsem)
    in_cps = [
        pltpu.async_copy(
            idx_hbm.at[pl.ds(base + c * CHUNK, CHUNK)],
            idx_v.at[pl.ds(c * CHUNK, CHUNK)],
            isem.at[c],
        )
        for c in range(NCHUNK)
    ]
    cp_t.wait()
    out_cps = []
    for c in range(NCHUNK):
        in_cps[c].wait()

        @plsc.parallel_loop(c * CHUNK, (c + 1) * CHUNK, step=LANES, unroll=8)
        def _(off):
            out_v[pl.ds(off, LANES)] = plsc.load_gather(
                tab_v, [idx_v[pl.ds(off, LANES)]]
            )

        out_cps.append(
            pltpu.async_copy(
                out_v.at[pl.ds(c * CHUNK, CHUNK)],
                out_hbm.at[pl.ds(base + c * CHUNK, CHUNK)],
                osem.at[c],
            )
        )
    for cp in out_cps:
        cp.wait()


def kernel(indices, table):
    out = _lookup(indices.astype(jnp.int32), table.reshape(2))
    return out.reshape(B, 1)
